# Initial kernel scaffold; baseline (speedup 1.0000x reference)
import jax
import jax.numpy as jnp
from jax.experimental import pallas as pl


def kernel(x, w0, b0, w3, b3, w5, b5, w6, b6, w8, b8):
    raise NotImplementedError("write your pallas kernel here")


# speedup vs baseline: 7.0380x; 7.0380x over previous
import functools

import numpy as np

import jax
import jax.numpy as jnp
from jax.experimental import pallas as pl
from jax.experimental.pallas import tpu as pltpu

_H = 16
_W = 16
_CIN = 4
_C0 = 8
_C3 = 16
_C5 = 8
_NCLS = 3


def _shift_dn(a):
    return jnp.concatenate([jnp.zeros_like(a[:, :1]), a[:, :-1]], axis=1)


def _shift_up(a):
    return jnp.concatenate([a[:, 1:], jnp.zeros_like(a[:, :1])], axis=1)


def _ext(g, fw):
    odd = g[:, :, fw:]
    even = g[:, :, :fw]
    return jnp.concatenate([_shift_dn(odd), g, _shift_up(even)], axis=2)


def _mm(a, m_ref, b_ref):
    B, R, K = a.shape
    n = m_ref.shape[1]
    y = jnp.dot(a.reshape(B * R, K), m_ref[...],
                preferred_element_type=jnp.float32)
    return y.reshape(B, R, n) + b_ref[...]


def _unet_kernel(x_ref, m0_ref, b0_ref, m3_ref, b3_ref, m5_ref, b5_ref,
                 m6_ref, b6_ref, m8_ref, b8_ref, out_ref, *, B):
    f32 = jnp.float32

    x = x_ref[...]
    ext_c = [
        _ext(x[:, c, :].reshape(B, _H // 2, 2 * _W), _W)
        for c in range(_CIN)
    ]
    x0 = jnp.concatenate(ext_c, axis=2)

    y0 = _mm(x0, m0_ref, b0_ref)
    a0 = jnp.maximum(y0, 0.0)

    pv = jnp.maximum(a0[:, :, :128], a0[:, :, 128:])
    pr = pv.reshape(B, 8, _C0, _W // 2, 2)
    pooled = jnp.maximum(pr[..., 0], pr[..., 1]).reshape(B, 8, 64)

    x3 = _ext(pooled.reshape(B, 4, 128), 64)
    a3 = jnp.maximum(_mm(x3, m3_ref, b3_ref), 0.0)

    a5 = a3.reshape(B, 8, 128)
    u = _mm(a5, m5_ref, b5_ref)

    x6 = jnp.concatenate([_ext(y0, 128), _ext(u, 128)], axis=2)
    a6 = jnp.maximum(_mm(x6, m6_ref, b6_ref), 0.0)

    o = _mm(a6, m8_ref, b8_ref)
    per_cls = [
        o[:, :, 32 * c:32 * (c + 1)].reshape(B, 1, 2 * _H * _W // 2)
        for c in range(_NCLS)
    ]
    out_ref[...] = jnp.concatenate(per_cls, axis=1).astype(f32)


def _band_matrix(w, width, kh_sel, kw_sel):
    m = jnp.einsum('oikl,rpk,wvl->irwpov', w, kh_sel, kw_sel)
    cin, cout = w.shape[1], w.shape[0]
    return m.reshape(4 * cin * width, 2 * cout * width)


def _band_matrix_rmajor(w, width, kh_sel, kw_sel):
    m = jnp.einsum('oikl,rpk,wvl->riwpov', w, kh_sel, kw_sel)
    cin, cout = w.shape[1], w.shape[0]
    return m.reshape(4 * cin * width, 2 * cout * width)


def _kh_selector():
    kh = np.zeros((4, 2, 3), np.float32)
    for r in range(4):
        for p in range(2):
            k = r - p
            if 0 <= k <= 2:
                kh[r, p, k] = 1.0
    return kh


def _kw_selector(width):
    kw = np.zeros((width, width, 3), np.float32)
    for wi in range(width):
        for wo in range(width):
            k = wi - wo + 1
            if 0 <= k <= 2:
                kw[wi, wo, k] = 1.0
    return kw


def kernel(x, w0, b0, w3, b3, w5, b5, w6, b6, w8, b8):
    n = x.shape[0]
    x4 = x.reshape(n, _CIN, _H * _W).astype(jnp.float32)

    kh = _kh_selector()
    kw16 = _kw_selector(_W)
    kw8 = _kw_selector(_W // 2)

    m0 = _band_matrix(w0, _W, kh, kw16)
    m3 = _band_matrix_rmajor(w3, _W // 2, kh, kw8)
    m6 = jnp.concatenate([
        _band_matrix_rmajor(w6[:, :_C0], _W, kh, kw16),
        _band_matrix_rmajor(w6[:, _C0:], _W, kh, kw16),
    ], axis=0)

    kw5 = np.zeros((8, 2, 16), np.float32)
    for wi in range(8):
        for t in range(2):
            kw5[wi, t, 2 * wi + t] = 1.0
    m5 = jnp.einsum('iokl,wlv->iwkov', w5, kw5).reshape(128, 256)

    a8 = w8[:, :, 0, 0]
    eye2 = np.eye(2, dtype=np.float32)
    eye16 = np.eye(16, dtype=np.float32)
    m8 = jnp.einsum('oi,pq,wv->piwoqv', a8, eye2, eye16).reshape(256, 96)

    def prow(b, width):
        return jnp.tile(jnp.repeat(b, width), 2)[None, :]

    b0r = prow(b0, _W)
    b3r = prow(b3, _W // 2)
    b5r = jnp.tile(jnp.repeat(b5, _W), 2)[None, :]
    b6r = prow(b6, _W)
    b8r = jnp.repeat(b8, 32)[None, :]

    bsz = min(32, n)
    assert n % bsz == 0
    grid = (n // bsz,)

    out = pl.pallas_call(
        functools.partial(_unet_kernel, B=bsz),
        out_shape=jax.ShapeDtypeStruct((n, _NCLS, _H * _W), jnp.float32),
        grid=grid,
        in_specs=[
            pl.BlockSpec((bsz, _CIN, _H * _W), lambda i: (i, 0, 0)),
            pl.BlockSpec(m0.shape, lambda i: (0, 0)),
            pl.BlockSpec(b0r.shape, lambda i: (0, 0)),
            pl.BlockSpec(m3.shape, lambda i: (0, 0)),
            pl.BlockSpec(b3r.shape, lambda i: (0, 0)),
            pl.BlockSpec(m5.shape, lambda i: (0, 0)),
            pl.BlockSpec(b5r.shape, lambda i: (0, 0)),
            pl.BlockSpec(m6.shape, lambda i: (0, 0)),
            pl.BlockSpec(b6r.shape, lambda i: (0, 0)),
            pl.BlockSpec(m8.shape, lambda i: (0, 0)),
            pl.BlockSpec(b8r.shape, lambda i: (0, 0)),
        ],
        out_specs=pl.BlockSpec((bsz, _NCLS, _H * _W), lambda i: (i, 0, 0)),
        compiler_params=pltpu.CompilerParams(
            dimension_semantics=("parallel",)),
    )(x4, m0, b0r, m3, b3r, m5, b5r, m6, b6r, m8, b8r)

    return out.reshape(n, _NCLS, _H, _W)


# speedup vs baseline: 29.2685x; 4.1586x over previous
import functools

import numpy as np

import jax
import jax.numpy as jnp
from jax.experimental import pallas as pl
from jax.experimental.pallas import tpu as pltpu

_H = 16
_W = 16
_CIN = 4
_C0 = 8
_C3 = 16
_C5 = 8
_NCLS = 3


def _shift_dn(a):
    return jnp.concatenate([jnp.zeros_like(a[:, :1]), a[:, :-1]], axis=1)


def _shift_up(a):
    return jnp.concatenate([a[:, 1:], jnp.zeros_like(a[:, :1])], axis=1)


def _ext(g, fw):
    odd = g[:, :, fw:]
    even = g[:, :, :fw]
    return jnp.concatenate([_shift_dn(odd), g, _shift_up(even)], axis=2)


def _mm(a, m_ref, b_ref):
    B, R, K = a.shape
    n = m_ref.shape[1]
    y = jnp.dot(a.reshape(B * R, K), m_ref[...],
                preferred_element_type=jnp.float32)
    return y.reshape(B, R, n) + b_ref[...]


def _unet_kernel(x_ref, m0_ref, b0_ref, m3_ref, b3_ref, m5_ref, b5_ref,
                 m6_ref, b6_ref, m8_ref, b8_ref, out_ref, *, B):
    f32 = jnp.float32

    x = x_ref[...]
    ext_c = [
        _ext(x[:, c, :].reshape(B, _H // 2, 2 * _W), _W)
        for c in range(_CIN)
    ]
    x0 = jnp.concatenate(ext_c, axis=2)

    y0 = _mm(x0, m0_ref, b0_ref)
    a0 = jnp.maximum(y0, 0.0)

    pv = jnp.maximum(a0[:, :, :128], a0[:, :, 128:])
    pooled = jnp.maximum(pv[:, :, :64], pv[:, :, 64:])

    x3 = jnp.concatenate([_shift_dn(pooled), pooled, _shift_up(pooled)],
                         axis=2)
    a5 = jnp.maximum(_mm(x3, m3_ref, b3_ref), 0.0)

    u = _mm(a5, m5_ref, b5_ref)

    x6 = jnp.concatenate([_ext(y0, 128), _ext(u, 128)], axis=2)
    a6 = jnp.maximum(_mm(x6, m6_ref, b6_ref), 0.0)

    o = _mm(a6, m8_ref, b8_ref)
    per_cls = [
        o[:, :, 32 * c:32 * (c + 1)].reshape(B, 1, 2 * _H * _W // 2)
        for c in range(_NCLS)
    ]
    out_ref[...] = jnp.concatenate(per_cls, axis=1).astype(f32)


def _band_matrix(w, width, kh_sel, kw_sel):
    m = jnp.einsum('oikl,rpk,wvl->irwpov', w, kh_sel, kw_sel)
    cin, cout = w.shape[1], w.shape[0]
    return m.reshape(4 * cin * width, 2 * cout * width)


def _band_matrix_rmajor(w, width, kh_sel, kw_sel):
    m = jnp.einsum('oikl,rpk,wvl->riwpov', w, kh_sel, kw_sel)
    cin, cout = w.shape[1], w.shape[0]
    return m.reshape(4 * cin * width, 2 * cout * width)


def _kh_selector():
    kh = np.zeros((4, 2, 3), np.float32)
    for r in range(4):
        for p in range(2):
            k = r - p
            if 0 <= k <= 2:
                kh[r, p, k] = 1.0
    return kh


def _kw_selector(width):
    kw = np.zeros((width, width, 3), np.float32)
    for wi in range(width):
        for wo in range(width):
            k = wi - wo + 1
            if 0 <= k <= 2:
                kw[wi, wo, k] = 1.0
    return kw


def kernel(x, w0, b0, w3, b3, w5, b5, w6, b6, w8, b8):
    n = x.shape[0]
    x4 = x.reshape(n, _CIN, _H * _W).astype(jnp.float32)

    kh = _kh_selector()
    kw16 = _kw_selector(_W)
    kw8 = _kw_selector(_W // 2)

    pool_perm = np.empty(128, np.int64)
    for par in range(2):
        for co in range(_C0):
            for j in range(_W // 2):
                pool_perm[par * 64 + co * 8 + j] = co * _W + 2 * j + par
    perm256 = np.concatenate([pool_perm, pool_perm + 128])

    m0 = _band_matrix(w0, _W, kh, kw16)[:, perm256]
    m3 = jnp.einsum('oikl,wvl->kiwov', w3, kw8).reshape(192, 128)
    perm512 = np.concatenate([pool_perm + 128 * r for r in range(4)])
    m6 = jnp.concatenate([
        _band_matrix_rmajor(w6[:, :_C0], _W, kh, kw16)[perm512, :],
        _band_matrix_rmajor(w6[:, _C0:], _W, kh, kw16),
    ], axis=0)

    kw5 = np.zeros((8, 2, 16), np.float32)
    for wi in range(8):
        for t in range(2):
            kw5[wi, t, 2 * wi + t] = 1.0
    m5 = jnp.einsum('iokl,wlv->iwkov', w5, kw5).reshape(128, 256)

    a8 = w8[:, :, 0, 0]
    eye2 = np.eye(2, dtype=np.float32)
    eye16 = np.eye(16, dtype=np.float32)
    m8 = jnp.einsum('oi,pq,wv->piwoqv', a8, eye2, eye16).reshape(256, 96)

    def prow(b, width):
        return jnp.tile(jnp.repeat(b, width), 2)[None, :]

    b0r = prow(b0, _W)[:, perm256]
    b3r = jnp.repeat(b3, _W // 2)[None, :]
    b5r = jnp.tile(jnp.repeat(b5, _W), 2)[None, :]
    b6r = prow(b6, _W)
    b8r = jnp.repeat(b8, 32)[None, :]

    bsz = min(64, n)
    assert n % bsz == 0
    grid = (n // bsz,)

    out = pl.pallas_call(
        functools.partial(_unet_kernel, B=bsz),
        out_shape=jax.ShapeDtypeStruct((n, _NCLS, _H * _W), jnp.float32),
        grid=grid,
        in_specs=[
            pl.BlockSpec((bsz, _CIN, _H * _W), lambda i: (i, 0, 0)),
            pl.BlockSpec(m0.shape, lambda i: (0, 0)),
            pl.BlockSpec(b0r.shape, lambda i: (0, 0)),
            pl.BlockSpec(m3.shape, lambda i: (0, 0)),
            pl.BlockSpec(b3r.shape, lambda i: (0, 0)),
            pl.BlockSpec(m5.shape, lambda i: (0, 0)),
            pl.BlockSpec(b5r.shape, lambda i: (0, 0)),
            pl.BlockSpec(m6.shape, lambda i: (0, 0)),
            pl.BlockSpec(b6r.shape, lambda i: (0, 0)),
            pl.BlockSpec(m8.shape, lambda i: (0, 0)),
            pl.BlockSpec(b8r.shape, lambda i: (0, 0)),
        ],
        out_specs=pl.BlockSpec((bsz, _NCLS, _H * _W), lambda i: (i, 0, 0)),
        compiler_params=pltpu.CompilerParams(
            dimension_semantics=("parallel",)),
    )(x4, m0, b0r, m3, b3r, m5, b5r, m6, b6r, m8, b8r)

    return out.reshape(n, _NCLS, _H, _W)


# speedup vs baseline: 60.5971x; 2.0704x over previous
import functools

import numpy as np

import jax
import jax.numpy as jnp
from jax.experimental import pallas as pl
from jax.experimental.pallas import tpu as pltpu

_H = 16
_W = 16
_CIN = 4
_C0 = 8
_C3 = 16
_C5 = 8
_NCLS = 3


def _shift_dn(a):
    return jnp.concatenate([jnp.zeros_like(a[:, :1]), a[:, :-1]], axis=1)


def _shift_up(a):
    return jnp.concatenate([a[:, 1:], jnp.zeros_like(a[:, :1])], axis=1)


def _ext(g, fw):
    odd = g[:, :, fw:]
    even = g[:, :, :fw]
    return jnp.concatenate([_shift_dn(odd), g, _shift_up(even)], axis=2)


def _mm(a, m_ref, b_ref):
    B, R, K = a.shape
    n = m_ref.shape[1]
    y = jnp.dot(a.reshape(B * R, K), m_ref[...],
                preferred_element_type=jnp.float32)
    return y.reshape(B, R, n) + b_ref[...]


def _unet_kernel(x_ref, m0_ref, b0_ref, m3_ref, b3_ref, m5_ref, b5_ref,
                 m6_ref, b6_ref, m8_ref, b8_ref, out_ref, *, B):
    f32 = jnp.float32

    x0 = _ext(x_ref[...], 64)

    y0 = _mm(x0, m0_ref, b0_ref)
    a0 = jnp.maximum(y0, 0.0)

    pv = jnp.maximum(a0[:, :, :128], a0[:, :, 128:])
    pooled = jnp.maximum(pv[:, :, :64], pv[:, :, 64:])

    x3 = jnp.concatenate([_shift_dn(pooled), pooled, _shift_up(pooled)],
                         axis=2)
    a5 = jnp.maximum(_mm(x3, m3_ref, b3_ref), 0.0)

    u = _mm(a5, m5_ref, b5_ref)

    x6 = jnp.concatenate([_ext(y0, 128), _ext(u, 128)], axis=2)
    a6 = jnp.maximum(_mm(x6, m6_ref, b6_ref), 0.0)

    out_ref[...] = _mm(a6, m8_ref, b8_ref).astype(f32)


def _band_matrix_rmajor(w, width, kh_sel, kw_sel):
    m = jnp.einsum('oikl,rpk,wvl->riwpov', w, kh_sel, kw_sel)
    cin, cout = w.shape[1], w.shape[0]
    return m.reshape(4 * cin * width, 2 * cout * width)


def _kh_selector():
    kh = np.zeros((4, 2, 3), np.float32)
    for r in range(4):
        for p in range(2):
            k = r - p
            if 0 <= k <= 2:
                kh[r, p, k] = 1.0
    return kh


def _kw_selector(width):
    kw = np.zeros((width, width, 3), np.float32)
    for wi in range(width):
        for wo in range(width):
            k = wi - wo + 1
            if 0 <= k <= 2:
                kw[wi, wo, k] = 1.0
    return kw


def kernel(x, w0, b0, w3, b3, w5, b5, w6, b6, w8, b8):
    n = x.shape[0]
    x4 = jnp.transpose(x.astype(jnp.float32), (0, 2, 1, 3)).reshape(
        n, _H // 2, 2 * _CIN * _W)

    kh = _kh_selector()
    kw16 = _kw_selector(_W)
    kw8 = _kw_selector(_W // 2)

    pool_perm = np.empty(128, np.int64)
    for par in range(2):
        for co in range(_C0):
            for j in range(_W // 2):
                pool_perm[par * 64 + co * 8 + j] = co * _W + 2 * j + par
    perm256 = np.concatenate([pool_perm, pool_perm + 128])

    m0 = _band_matrix_rmajor(w0, _W, kh, kw16)[:, perm256]
    m3 = jnp.einsum('oikl,wvl->kiwov', w3, kw8).reshape(192, 128)
    perm512 = np.concatenate([pool_perm + 128 * r for r in range(4)])
    m6 = jnp.concatenate([
        _band_matrix_rmajor(w6[:, :_C0], _W, kh, kw16)[perm512, :],
        _band_matrix_rmajor(w6[:, _C0:], _W, kh, kw16),
    ], axis=0)

    kw5 = np.zeros((8, 2, 16), np.float32)
    for wi in range(8):
        for t in range(2):
            kw5[wi, t, 2 * wi + t] = 1.0
    m5 = jnp.einsum('iokl,wlv->iwkov', w5, kw5).reshape(128, 256)

    a8 = w8[:, :, 0, 0]
    eye2 = np.eye(2, dtype=np.float32)
    eye16 = np.eye(16, dtype=np.float32)
    m8 = jnp.einsum('oi,pq,wv->piwoqv', a8, eye2, eye16).reshape(256, 96)

    def prow(b, width):
        return jnp.tile(jnp.repeat(b, width), 2)[None, :]

    b0r = prow(b0, _W)[:, perm256]
    b3r = jnp.repeat(b3, _W // 2)[None, :]
    b5r = jnp.tile(jnp.repeat(b5, _W), 2)[None, :]
    b6r = prow(b6, _W)
    b8r = jnp.repeat(b8, 32)[None, :]

    bsz = min(64, n)
    assert n % bsz == 0
    grid = (n // bsz,)

    out = pl.pallas_call(
        functools.partial(_unet_kernel, B=bsz),
        out_shape=jax.ShapeDtypeStruct((n, _H // 2, 2 * _NCLS * _W),
                                       jnp.float32),
        grid=grid,
        in_specs=[
            pl.BlockSpec((bsz, _H // 2, 2 * _CIN * _W), lambda i: (i, 0, 0)),
            pl.BlockSpec(m0.shape, lambda i: (0, 0)),
            pl.BlockSpec(b0r.shape, lambda i: (0, 0)),
            pl.BlockSpec(m3.shape, lambda i: (0, 0)),
            pl.BlockSpec(b3r.shape, lambda i: (0, 0)),
            pl.BlockSpec(m5.shape, lambda i: (0, 0)),
            pl.BlockSpec(b5r.shape, lambda i: (0, 0)),
            pl.BlockSpec(m6.shape, lambda i: (0, 0)),
            pl.BlockSpec(b6r.shape, lambda i: (0, 0)),
            pl.BlockSpec(m8.shape, lambda i: (0, 0)),
            pl.BlockSpec(b8r.shape, lambda i: (0, 0)),
        ],
        out_specs=pl.BlockSpec((bsz, _H // 2, 2 * _NCLS * _W),
                               lambda i: (i, 0, 0)),
        compiler_params=pltpu.CompilerParams(
            dimension_semantics=("parallel",)),
    )(x4, m0, b0r, m3, b3r, m5, b5r, m6, b6r, m8, b8r)

    return jnp.transpose(out.reshape(n, _H // 2, _NCLS, 2, _W),
                         (0, 2, 1, 3, 4)).reshape(n, _NCLS, _H, _W)
